# baseline (device time: 87910 ns/iter reference)
import jax
import jax.numpy as jnp
from jax import lax
from jax.experimental import pallas as pl
from jax.experimental.pallas import tpu as pltpu

N_DEV = 16
N_Z = 4
N_W = 4


def kernel(x, w_mat, scale_x, scale_w):
    m_per, k = x.shape
    _, n_per = w_mat.shape
    m_c = 192
    m_ch = 96
    m_p = 64
    m_ph = 32
    m_u = 3 * m_p

    def body(x_ref, w_ref, sx_ref, sw_ref, out_ref,
             col_buf, full_cw, full_ccw, half_a, half_b,
             p_cw, p_ccw, p_ha, p_hb, punit,
             zs_up, zr_up, zs_dn, zr_dn,
             s_cwf, r_cwf, s_ccwf, r_ccwf,
             s_cwh, r_cwh, s_ccwh, r_ccwh,
             ps_cwf, pr_cwf, ps_ccwf, pr_ccwf,
             ps_cwh, pr_cwh, ps_ccwh, pr_ccwh,
             ps_up, pr_up, ps_dn, pr_dn):
        my = lax.axis_index("i")
        z = lax.div(my, N_W)
        w = lax.rem(my, N_W)
        cw_dev = N_W * z + lax.rem(w + 1, N_W)
        ccw_dev = N_W * z + lax.rem(w + 3, N_W)
        up_dev = N_W * jnp.minimum(z + 1, N_Z - 1) + w
        dn_dev = N_W * jnp.maximum(z - 1, 0) + w
        has_up = z < N_Z - 1
        has_dn = z > 0

        barrier_sem = pltpu.get_barrier_semaphore()
        for tgt in (cw_dev, ccw_dev):
            pl.semaphore_signal(
                barrier_sem, inc=1,
                device_id=(tgt,), device_id_type=pl.DeviceIdType.MESH,
            )

        @pl.when(has_up)
        def _():
            pl.semaphore_signal(
                barrier_sem, inc=1,
                device_id=(up_dev,), device_id_type=pl.DeviceIdType.MESH,
            )

        @pl.when(has_dn)
        def _():
            pl.semaphore_signal(
                barrier_sem, inc=1,
                device_id=(dn_dev,), device_id_type=pl.DeviceIdType.MESH,
            )

        both_z = jnp.logical_and(has_up, has_dn)

        @pl.when(both_z)
        def _():
            pl.semaphore_wait(barrier_sem, 4)

        @pl.when(jnp.logical_not(both_z))
        def _():
            pl.semaphore_wait(barrier_sem, 3)

        scale = sx_ref[0] * sw_ref[0]

        def gemm_store(chunk, origin, row_off, rows):
            acc = lax.dot_general(
                chunk, w_ref[...],
                (((1,), (0,)), ((), ())),
                preferred_element_type=jnp.int32,
            )
            out_ref[pl.ds(origin * m_per + row_off, rows), :] = (
                acc.astype(jnp.float32) * scale
            )

        def rc(src, dst, ssem, rsem, dev):
            return pltpu.make_async_remote_copy(
                src_ref=src, dst_ref=dst, send_sem=ssem, recv_sem=rsem,
                device_id=(dev,), device_id_type=pl.DeviceIdType.MESH,
            )

        zup = [rc(col_buf.at[s], col_buf.at[s],
                  zs_up.at[s], zr_up.at[s], up_dev) for s in range(N_Z)]
        zupo = [rc(x_ref, col_buf.at[s],
                   zs_up.at[s], zr_up.at[s], up_dev) for s in range(N_Z)]
        zdn = [rc(col_buf.at[s], col_buf.at[s],
                  zs_dn.at[s], zr_dn.at[s], dn_dev) for s in range(N_Z)]
        zdno = [rc(x_ref, col_buf.at[s],
                   zs_dn.at[s], zr_dn.at[s], dn_dev) for s in range(N_Z)]
        cwf = [rc(col_buf.at[s, pl.ds(0, m_c), :], full_cw.at[s],
                  s_cwf.at[s], r_cwf.at[s], cw_dev) for s in range(N_Z)]
        cwfo = [rc(x_ref.at[pl.ds(0, m_c), :], full_cw.at[s],
                   s_cwf.at[s], r_cwf.at[s], cw_dev) for s in range(N_Z)]
        ccwf = [rc(col_buf.at[s, pl.ds(0, m_c), :], full_ccw.at[s],
                   s_ccwf.at[s], r_ccwf.at[s], ccw_dev) for s in range(N_Z)]
        ccwfo = [rc(x_ref.at[pl.ds(0, m_c), :], full_ccw.at[s],
                    s_ccwf.at[s], r_ccwf.at[s], ccw_dev) for s in range(N_Z)]
        cwh = [rc(full_cw.at[s, pl.ds(0, m_ch), :], half_a.at[s],
                  s_cwh.at[s], r_cwh.at[s], cw_dev) for s in range(N_Z)]
        ccwh = [rc(full_ccw.at[s, pl.ds(m_ch, m_ch), :], half_b.at[s],
                   s_ccwh.at[s], r_ccwh.at[s], ccw_dev) for s in range(N_Z)]
        pcwf = rc(x_ref.at[pl.ds(m_c, m_p), :], p_cw,
                  ps_cwf.at[0], pr_cwf.at[0], cw_dev)
        pccwf = rc(x_ref.at[pl.ds(m_c, m_p), :], p_ccw,
                   ps_ccwf.at[0], pr_ccwf.at[0], ccw_dev)
        pcwh = rc(p_cw.at[pl.ds(0, m_ph), :], p_ha,
                  ps_cwh.at[0], pr_cwh.at[0], cw_dev)
        pccwh = rc(p_ccw.at[pl.ds(m_ph, m_ph), :], p_hb,
                   ps_ccwh.at[0], pr_ccwh.at[0], ccw_dev)
        zupP = [rc(punit.at[s], punit.at[s],
                   ps_up.at[s], pr_up.at[s], up_dev) for s in range(N_Z)]
        zdnP = [rc(punit.at[s], punit.at[s],
                   ps_dn.at[s], pr_dn.at[s], dn_dev) for s in range(N_Z)]

        for s in range(N_Z):
            mine = z == s

            @pl.when(mine)
            def _(s=s):
                cwfo[s].start()
                ccwfo[s].start()

            @pl.when(jnp.logical_and(mine, has_up))
            def _(s=s):
                zupo[s].start()

            @pl.when(jnp.logical_and(mine, has_dn))
            def _(s=s):
                zdno[s].start()

        pcwf.start()
        pccwf.start()
        gemm_store(x_ref[...], my, 0, m_per)

        pcwf.wait_recv()
        pcwh.start()
        pccwf.wait_recv()
        pccwh.start()
        for s in range(N_Z):
            @pl.when(z == s)
            def _(s=s):
                punit[s, 0:m_p, :] = p_cw[...]
                punit[s, m_p:2 * m_p, :] = p_ccw[...]

        gemm_store(p_cw[...], N_W * z + lax.rem(w + 3, N_W), m_c, m_p)
        gemm_store(p_ccw[...], N_W * z + lax.rem(w + 1, N_W), m_c, m_p)

        def col_round(r):
            for s in range(N_Z - 1):
                pred = z == s + r

                @pl.when(pred)
                def _(s=s):
                    zup[s].wait_recv()

                @pl.when(jnp.logical_and(pred, has_up))
                def _(s=s):
                    zup[s].start()

                @pl.when(pred)
                def _(s=s):
                    cwf[s].start()
                    ccwf[s].start()
                    gemm_store(col_buf[s], N_W * s + w, 0, m_per)

            for s in range(1, N_Z):
                pred = z == s - r

                @pl.when(pred)
                def _(s=s):
                    zdn[s].wait_recv()

                @pl.when(jnp.logical_and(pred, has_dn))
                def _(s=s):
                    zdn[s].start()

                @pl.when(pred)
                def _(s=s):
                    cwf[s].start()
                    ccwf[s].start()
                    gemm_store(col_buf[s], N_W * s + w, 0, m_per)

        def punit_gemms(src, s_expr):
            gemm_store(src[0:m_p],
                       N_W * s_expr + lax.rem(w + 3, N_W), m_c, m_p)
            gemm_store(src[m_p:2 * m_p],
                       N_W * s_expr + lax.rem(w + 1, N_W), m_c, m_p)
            gemm_store(src[2 * m_p:3 * m_p],
                       N_W * s_expr + lax.rem(w + 2, N_W), m_c, m_p)

        def p_round(r):
            for s in range(N_Z - 1):
                pred = z == s + r

                @pl.when(pred)
                def _(s=s):
                    zupP[s].wait_recv()

                @pl.when(jnp.logical_and(pred, has_up))
                def _(s=s):
                    zupP[s].start()

                @pl.when(pred)
                def _(s=s):
                    punit_gemms(punit[s], s)

            for s in range(1, N_Z):
                pred = z == s - r

                @pl.when(pred)
                def _(s=s):
                    zdnP[s].wait_recv()

                @pl.when(jnp.logical_and(pred, has_dn))
                def _(s=s):
                    zdnP[s].start()

                @pl.when(pred)
                def _(s=s):
                    punit_gemms(punit[s], s)

        col_round(1)

        pcwh.wait_recv()
        pccwh.wait_recv()
        for s in range(N_Z):
            mine = z == s

            @pl.when(mine)
            def _(s=s):
                punit[s, 2 * m_p:2 * m_p + m_ph, :] = p_ha[...]
                punit[s, 2 * m_p + m_ph:m_u, :] = p_hb[...]

            @pl.when(jnp.logical_and(mine, has_up))
            def _(s=s):
                zupP[s].start()

            @pl.when(jnp.logical_and(mine, has_dn))
            def _(s=s):
                zdnP[s].start()

        diag_me = N_W * z + lax.rem(w + 2, N_W)
        gemm_store(p_ha[...], diag_me, m_c, m_ph)
        gemm_store(p_hb[...], diag_me, m_c + m_ph, m_ph)

        col_round(2)
        col_round(3)
        p_round(1)

        def square_fulls(s):
            cwf[s].wait_recv()
            cwh[s].start()
            gemm_store(full_cw[s], N_W * s + lax.rem(w + 3, N_W), 0, m_c)
            ccwf[s].wait_recv()
            ccwh[s].start()
            gemm_store(full_ccw[s], N_W * s + lax.rem(w + 1, N_W), 0, m_c)

        square_fulls(0)
        square_fulls(1)
        p_round(2)
        square_fulls(2)
        square_fulls(3)
        p_round(3)

        for s in range(N_Z):
            diag = N_W * s + lax.rem(w + 2, N_W)
            cwh[s].wait_recv()
            gemm_store(half_a[s], diag, 0, m_ch)
            ccwh[s].wait_recv()
            gemm_store(half_b[s], diag, m_ch, m_ch)

        for s in range(N_Z):
            @pl.when(jnp.logical_and(s <= z, has_up))
            def _(s=s):
                zup[s].wait_send()
                zupP[s].wait_send()

            @pl.when(jnp.logical_and(s >= z, has_dn))
            def _(s=s):
                zdn[s].wait_send()
                zdnP[s].wait_send()

            cwf[s].wait_send()
            ccwf[s].wait_send()
            cwh[s].wait_send()
            ccwh[s].wait_send()

        for r in (pcwf, pccwf, pcwh, pccwh):
            r.wait_send()

    return pl.pallas_call(
        body,
        out_shape=jax.ShapeDtypeStruct((N_DEV * m_per, n_per), jnp.float32),
        in_specs=[
            pl.BlockSpec(memory_space=pltpu.VMEM),
            pl.BlockSpec(memory_space=pltpu.VMEM),
            pl.BlockSpec(memory_space=pltpu.SMEM),
            pl.BlockSpec(memory_space=pltpu.SMEM),
        ],
        out_specs=pl.BlockSpec(memory_space=pltpu.VMEM),
        scratch_shapes=[
            pltpu.VMEM((N_Z, m_per, k), jnp.int8),
            pltpu.VMEM((N_Z, m_c, k), jnp.int8),
            pltpu.VMEM((N_Z, m_c, k), jnp.int8),
            pltpu.VMEM((N_Z, m_ch, k), jnp.int8),
            pltpu.VMEM((N_Z, m_ch, k), jnp.int8),
            pltpu.VMEM((m_p, k), jnp.int8),
            pltpu.VMEM((m_p, k), jnp.int8),
            pltpu.VMEM((m_ph, k), jnp.int8),
            pltpu.VMEM((m_ph, k), jnp.int8),
            pltpu.VMEM((N_Z, m_u, k), jnp.int8),
        ]
        + [pltpu.SemaphoreType.DMA((N_Z,))] * 12
        + [pltpu.SemaphoreType.DMA((1,))] * 8
        + [pltpu.SemaphoreType.DMA((N_Z,))] * 4,
        compiler_params=pltpu.CompilerParams(collective_id=0),
    )(x, w_mat, scale_x, scale_w)
